# Initial kernel scaffold; baseline (speedup 1.0000x reference)
import functools

import jax
import jax.numpy as jnp
from jax import lax
from jax.experimental import pallas as pl
from jax.experimental.pallas import tpu as pltpu
from jax.experimental.pallas import tpu_sc as plsc

_NC = 2
_NS = 16
_NW = _NC * _NS
_L = 16
_C = 400


@functools.partial(jax.jit, static_argnames=("n_edges", "d"))
def _sc_edge_dot(h, src, dst, *, n_edges, d):
    per_w = n_edges // _NW
    n_chunks = per_w // _C
    mesh = plsc.VectorSubcoreMesh(core_axis_name="c", subcore_axis_name="s")

    @functools.partial(
        pl.kernel,
        out_type=jax.ShapeDtypeStruct((n_edges,), jnp.float32),
        mesh=mesh,
        scratch_types=[
            pltpu.VMEM((_C,), jnp.int32),
            pltpu.VMEM((_C,), jnp.int32),
            pltpu.VMEM((_C, d), jnp.float32),
            pltpu.VMEM((_C, d), jnp.float32),
            pltpu.VMEM((_C,), jnp.float32),
            pltpu.SemaphoreType.DMA,
            pltpu.SemaphoreType.DMA,
        ],
    )
    def k(h_hbm, src_hbm, dst_hbm, out_hbm,
          sidx_v, didx_v, srow_v, drow_v, out_v, sem1, sem2):
        wid = lax.axis_index("s") * _NC + lax.axis_index("c")
        base_w = wid * per_w

        @pl.loop(0, n_chunks)
        def _(i):
            base = base_w + i * _C
            pltpu.sync_copy(src_hbm.at[pl.ds(base, _C)], sidx_v)
            pltpu.sync_copy(dst_hbm.at[pl.ds(base, _C)], didx_v)
            cp1 = pltpu.async_copy(h_hbm.at[sidx_v], srow_v, sem1)
            cp2 = pltpu.async_copy(h_hbm.at[didx_v], drow_v, sem2)
            cp1.wait()
            cp2.wait()

            @pl.loop(0, _C)
            def _(r):
                acc = srow_v[r, pl.ds(0, _L)] * drow_v[r, pl.ds(0, _L)]
                for kk in range(1, d // _L):
                    acc = acc + (srow_v[r, pl.ds(kk * _L, _L)]
                                 * drow_v[r, pl.ds(kk * _L, _L)])
                out_v[r] = jnp.sum(acc)

            pltpu.sync_copy(out_v, out_hbm.at[pl.ds(base, _C)])

    return k(h, src, dst)


def kernel(h, edge_index):
    n_nodes, d = h.shape
    n_edges = edge_index.shape[1]
    assert n_edges % (_NW * _C) == 0 and d % _L == 0
    src = edge_index[0].astype(jnp.int32)
    dst = edge_index[1].astype(jnp.int32)
    score = _sc_edge_dot(h, src, dst, n_edges=n_edges, d=d)
    return score.reshape(n_edges, 1)


# speedup vs baseline: 1.1984x; 1.1984x over previous
import dataclasses
import functools

import jax
import jax.numpy as jnp
from jax import lax
from jax.experimental import pallas as pl
from jax.experimental.pallas import tpu as pltpu
from jax.experimental.pallas import tpu_sc as plsc

_NC = 2
_NS = 16
_NW = _NC * _NS
_L = 16
_C = 400


@functools.partial(jax.jit, static_argnames=("n_edges", "d"))
def _sc_edge_dot(h, src, dst, *, n_edges, d):
    per_w = n_edges // _NW
    n_chunks = per_w // _C
    mesh = plsc.VectorSubcoreMesh(core_axis_name="c", subcore_axis_name="s")
    cp = pltpu.CompilerParams()
    if "needs_layout_passes" in pltpu.CompilerParams.__dataclass_fields__:
        cp = dataclasses.replace(cp, needs_layout_passes=False)

    @functools.partial(
        pl.kernel,
        compiler_params=cp,
        out_type=jax.ShapeDtypeStruct((n_edges,), jnp.float32),
        mesh=mesh,
        scratch_types=[
            pltpu.VMEM((_C,), jnp.int32),
            pltpu.VMEM((_C,), jnp.int32),
            pltpu.VMEM((_C, d), jnp.float32),
            pltpu.VMEM((_C, d), jnp.float32),
            pltpu.VMEM((_C,), jnp.float32),
            pltpu.SemaphoreType.DMA,
            pltpu.SemaphoreType.DMA,
        ],
    )
    def k(h_hbm, src_hbm, dst_hbm, out_hbm,
          sidx_v, didx_v, srow_v, drow_v, out_v, sem1, sem2):
        wid = lax.axis_index("s") * _NC + lax.axis_index("c")
        base_w = wid * per_w

        @pl.loop(0, n_chunks)
        def _(i):
            base = base_w + i * _C
            pltpu.sync_copy(src_hbm.at[pl.ds(base, _C)], sidx_v)
            pltpu.sync_copy(dst_hbm.at[pl.ds(base, _C)], didx_v)
            cp1 = pltpu.async_copy(h_hbm.at[sidx_v], srow_v, sem1)
            cp2 = pltpu.async_copy(h_hbm.at[didx_v], drow_v, sem2)
            cp1.wait()
            cp2.wait()

            @pl.loop(0, _C // _L)
            def _(g):
                row_idx = g * _L + lax.iota(jnp.int32, _L)

                def body(dd, acc):
                    col = jnp.full((_L,), dd, jnp.int32)
                    sv = plsc.load_gather(srow_v, [row_idx, col])
                    dv = plsc.load_gather(drow_v, [row_idx, col])
                    return acc + sv * dv

                acc = lax.fori_loop(0, d, body, jnp.zeros((_L,), jnp.float32))
                out_v[pl.ds(g * _L, _L)] = acc

            pltpu.sync_copy(out_v, out_hbm.at[pl.ds(base, _C)])

    return k(h, src, dst)


def kernel(h, edge_index):
    n_nodes, d = h.shape
    n_edges = edge_index.shape[1]
    assert n_edges % (_NW * _C) == 0 and d % _L == 0
    src = edge_index[0].astype(jnp.int32)
    dst = edge_index[1].astype(jnp.int32)
    score = _sc_edge_dot(h, src, dst, n_edges=n_edges, d=d)
    return score.reshape(n_edges, 1)


# speedup vs baseline: 4.2663x; 3.5599x over previous
import dataclasses
import functools

import jax
import jax.numpy as jnp
from jax import lax
from jax.experimental import pallas as pl
from jax.experimental.pallas import tpu as pltpu
from jax.experimental.pallas import tpu_sc as plsc

_NC = 2
_NS = 16
_L = 16
_C = 1280


@functools.partial(jax.jit, static_argnames=("n_edges", "d", "n_nodes"))
def _sc_edge_dot(ht, src, dst, *, n_edges, d, n_nodes):
    per_sc = n_edges // _NC
    n_chunks = per_sc // _C
    npairs = (n_chunks - 1) // 2
    assert n_chunks == 2 * npairs + 1
    nf = d // _NS
    sub = _C // _NS
    spg = sub // _L
    assert sub % _L == 0

    mesh = plsc.VectorSubcoreMesh(core_axis_name="c", subcore_axis_name="s")
    cp = pltpu.CompilerParams()
    if "needs_layout_passes" in pltpu.CompilerParams.__dataclass_fields__:
        cp = dataclasses.replace(cp, needs_layout_passes=False)

    @functools.partial(
        pl.kernel,
        compiler_params=cp,
        out_type=jax.ShapeDtypeStruct((n_edges,), jnp.float32),
        mesh=mesh,
        scratch_types=[
            pltpu.VMEM((nf, n_nodes), jnp.float32),
            pltpu.VMEM((_C,), jnp.int32),
            pltpu.VMEM((_C,), jnp.int32),
            pltpu.VMEM((_C,), jnp.int32),
            pltpu.VMEM((_C,), jnp.int32),
            pltpu.VMEM((_NS, 128), jnp.float32),
            pltpu.VMEM((_NS, 128), jnp.float32),
            pltpu.VMEM((sub,), jnp.float32),
            pltpu.VMEM_SHARED((2, _NS, _NS, 128), jnp.float32),
            pltpu.SemaphoreType.DMA,
            pltpu.SemaphoreType.DMA,
            pltpu.SemaphoreType.DMA,
            pltpu.SemaphoreType.DMA,
            pltpu.SemaphoreType.DMA,
        ],
    )
    def k(ht_hbm, src_hbm, dst_hbm, out_hbm,
          hsl, si0, di0, si1, di1, part_v, red_v, outb_v, ex_sh,
          hs_sem, is0, id0, is1, id1):
        cid = lax.axis_index("c")
        tid = lax.axis_index("s")
        base_sc = cid * per_sc

        cph = pltpu.make_async_copy(
            ht_hbm.at[pl.ds(tid * nf, nf)], hsl, hs_sem)
        cph.start()

        def idx_start(c, si, di, ssem, dsem):
            pltpu.make_async_copy(
                src_hbm.at[pl.ds(base_sc + c * _C, _C)], si, ssem).start()
            pltpu.make_async_copy(
                dst_hbm.at[pl.ds(base_sc + c * _C, _C)], di, dsem).start()

        def idx_wait(c, si, di, ssem, dsem):
            pltpu.make_async_copy(
                src_hbm.at[pl.ds(base_sc + c * _C, _C)], si, ssem).wait()
            pltpu.make_async_copy(
                dst_hbm.at[pl.ds(base_sc + c * _C, _C)], di, dsem).wait()

        idx_start(0, si0, di0, is0, id0)
        cph.wait()

        def body(c, slot, si, di):
            @pl.loop(0, _NS)
            def _(p):
                for j in range(spg):
                    off = p * sub + j * _L
                    s16 = si[pl.ds(off, _L)]
                    d16 = di[pl.ds(off, _L)]
                    acc0 = jnp.zeros((_L,), jnp.float32)
                    acc1 = jnp.zeros((_L,), jnp.float32)
                    for f in range(nf):
                        row = jnp.full((_L,), f, jnp.int32)
                        prod = (plsc.load_gather(hsl, [row, s16])
                                * plsc.load_gather(hsl, [row, d16]))
                        if f % 2 == 0:
                            acc0 = acc0 + prod
                        else:
                            acc1 = acc1 + prod
                    part_v[p, pl.ds(j * _L, _L)] = acc0 + acc1

            pltpu.sync_copy(part_v, ex_sh.at[slot, tid])
            plsc.subcore_barrier()
            pltpu.sync_copy(ex_sh.at[slot, :, tid], red_v)

            for j in range(spg):
                acc = red_v[0, pl.ds(j * _L, _L)]
                for r in range(1, _NS):
                    acc = acc + red_v[r, pl.ds(j * _L, _L)]
                outb_v[pl.ds(j * _L, _L)] = acc

            pltpu.sync_copy(
                outb_v,
                out_hbm.at[pl.ds(base_sc + c * _C + tid * sub, sub)])

        @pl.loop(0, npairs)
        def _(i):
            c0 = 2 * i
            idx_start(c0 + 1, si1, di1, is1, id1)
            idx_wait(c0, si0, di0, is0, id0)
            body(c0, 0, si0, di0)
            idx_start(c0 + 2, si0, di0, is0, id0)
            idx_wait(c0 + 1, si1, di1, is1, id1)
            body(c0 + 1, 1, si1, di1)

        idx_wait(n_chunks - 1, si0, di0, is0, id0)
        body(n_chunks - 1, 0, si0, di0)

    return k(ht, src, dst)


def kernel(h, edge_index):
    n_nodes, d = h.shape
    n_edges = edge_index.shape[1]
    assert n_edges % (_NC * _C) == 0 and d % _NS == 0
    ht = h.T
    src = edge_index[0].astype(jnp.int32)
    dst = edge_index[1].astype(jnp.int32)
    score = _sc_edge_dot(ht, src, dst, n_edges=n_edges, d=d, n_nodes=n_nodes)
    return score.reshape(n_edges, 1)
